# baseline (speedup 1.0000x reference)
import functools

import jax
import jax.numpy as jnp
from jax import lax
from jax.experimental import pallas as pl
from jax.experimental.pallas import tpu as pltpu
from jax.experimental.pallas import tpu_sc as plsc

N_NODES = 10000
D = 128
NC = 2
NS = 16
NW = NC * NS
CHUNK = 125
IDXG = 16
ACC_ROWS = N_NODES


def _make_seg_kernel(nch: int):
    mesh = plsc.VectorSubcoreMesh(core_axis_name="c", subcore_axis_name="s")

    @functools.partial(
        pl.kernel,
        out_type=jax.ShapeDtypeStruct((NC, N_NODES, D), jnp.float32),
        mesh=mesh,
        scratch_types=[
            pltpu.VMEM((IDXG, CHUNK), jnp.int32),
            pltpu.VMEM((IDXG, CHUNK), jnp.int32),
            pltpu.VMEM((2, CHUNK, D), jnp.float32),
            pltpu.VMEM_SHARED((ACC_ROWS, D), jnp.float32),
            pltpu.SemaphoreType.DMA,
            pltpu.SemaphoreType.DMA,
            pltpu.SemaphoreType.DMA,
            pltpu.SemaphoreType.DMA,
        ],
    )
    def seg(x_hbm, edges_hbm, out_hbm, src_v, dst_v, rows_v, acc, sem0,
            sem1, ssem0, ssem1):
        cid = lax.axis_index("c")
        sid = lax.axis_index("s")
        wid = cid * NS + sid

        def zrow(r, carry):
            for c8 in range(D // 16):
                rows_v[0, r, pl.ds(c8 * 16, 16)] = jnp.zeros((16,), jnp.float32)
            return carry

        lax.fori_loop(0, CHUNK, zrow, 0)
        zpt = ACC_ROWS // NS
        for k in range(zpt // CHUNK):
            pltpu.sync_copy(rows_v.at[0],
                            acc.at[pl.ds(sid * zpt + k * CHUNK, CHUNK)])
        plsc.subcore_barrier()

        def start_gather(i, b, sem):
            pltpu.async_copy(x_hbm.at[src_v.at[i]], rows_v.at[b], sem)

        def wait_gather(i, b, sem):
            pltpu.make_async_copy(x_hbm.at[src_v.at[i]], rows_v.at[b], sem).wait()

        def start_scatter(i, b, sem):
            pltpu.async_copy(rows_v.at[b], acc.at[dst_v.at[i]], sem, add=True)

        def wait_scatter(i, b, sem):
            pltpu.make_async_copy(rows_v.at[b], acc.at[dst_v.at[i]], sem).wait()

        def group(g, carry):
            base = wid * nch + g * IDXG
            pltpu.sync_copy(edges_hbm.at[0, pl.ds(base, IDXG)], src_v)
            pltpu.sync_copy(edges_hbm.at[1, pl.ds(base, IDXG)], dst_v)
            start_gather(0, 0, sem0)
            start_gather(1, 1, sem1)

            def pair(k, c2):
                i0 = 2 * k
                wait_gather(i0, 0, sem0)
                start_scatter(i0, 0, ssem0)
                wait_gather(i0 + 1, 1, sem1)
                start_scatter(i0 + 1, 1, ssem1)

                @pl.when(k < IDXG // 2 - 1)
                def _():
                    wait_scatter(i0, 0, ssem0)
                    start_gather(i0 + 2, 0, sem0)
                    wait_scatter(i0 + 1, 1, ssem1)
                    start_gather(i0 + 3, 1, sem1)

                return c2

            lax.fori_loop(0, IDXG // 2, pair, 0)
            wait_scatter(IDXG - 2, 0, ssem0)
            wait_scatter(IDXG - 1, 1, ssem1)
            return carry

        lax.fori_loop(0, nch // IDXG, group, 0)
        plsc.subcore_barrier()

        rpt = 624
        tail = N_NODES - (NS - 1) * rpt

        @pl.when(sid != NS - 1)
        def _():
            pltpu.sync_copy(acc.at[pl.ds(sid * rpt, rpt)],
                            out_hbm.at[cid, pl.ds(sid * rpt, rpt)])

        @pl.when(sid == NS - 1)
        def _():
            pltpu.sync_copy(acc.at[pl.ds((NS - 1) * rpt, tail)],
                            out_hbm.at[cid, pl.ds((NS - 1) * rpt, tail)])

    return seg


def _dense_mlp(partial, x, W_conv, W_self, bc2, W1, b12, W2, b22, W3p, b3p):
    RB = 1000
    grid = (N_NODES // RB,)

    def body(p_ref, x_ref, wc, ws, bc_r, w1, b1_r, w2, b2_r, w3, b3_r, o_ref):
        agg = p_ref[0] + p_ref[1]
        h = agg @ wc[...] + x_ref[...] @ ws[...] + bc_r[...]
        h = jnp.maximum(h, 0.0)
        h = jnp.maximum(h @ w1[...] + b1_r[...], 0.0)
        h = jnp.maximum(h @ w2[...] + b2_r[...], 0.0)
        z = jnp.maximum(h @ w3[...] + b3_r[...], 0.0)
        o_ref[...] = jax.nn.sigmoid(z[:, :1])

    full = lambda i: (0, 0)
    return pl.pallas_call(
        body,
        grid=grid,
        in_specs=[
            pl.BlockSpec((NC, RB, D), lambda i: (0, i, 0)),
            pl.BlockSpec((RB, D), lambda i: (i, 0)),
            pl.BlockSpec((D, D), full),
            pl.BlockSpec((D, D), full),
            pl.BlockSpec((1, D), full),
            pl.BlockSpec((D, D), full),
            pl.BlockSpec((1, D), full),
            pl.BlockSpec((D, D), full),
            pl.BlockSpec((1, D), full),
            pl.BlockSpec((D, D), full),
            pl.BlockSpec((1, D), full),
        ],
        out_specs=pl.BlockSpec((RB, 1), lambda i: (i, 0)),
        out_shape=jax.ShapeDtypeStruct((N_NODES, 1), jnp.float32),
    )(partial, x, W_conv, W_self, bc2, W1, b12, W2, b22, W3p, b3p)


def kernel(x, edge_index, W_conv, W_self, b_conv, W1, b1, W2, b2, W3, b3):
    E = edge_index.shape[1]

    assert E % (NW * CHUNK) == 0, "edge count must divide into 125-edge chunks"
    nch = E // (NW * CHUNK)
    assert nch % IDXG == 0
    edges = edge_index.astype(jnp.int32).reshape(2, NW * nch, CHUNK)

    partial = _make_seg_kernel(nch)(x, edges)

    bc2 = b_conv.reshape(1, D)
    b12 = b1.reshape(1, D)
    b22 = b2.reshape(1, D)
    W3p = jnp.pad(W3, ((0, 0), (0, D - W3.shape[1])))
    b3p = jnp.pad(b3.reshape(1, 1), ((0, 0), (0, D - 1)))

    out = _dense_mlp(partial, x, W_conv, W_self, bc2, W1, b12, W2, b22, W3p, b3p)
    return out.reshape(N_NODES)


# speedup vs baseline: 1.2737x; 1.2737x over previous
import functools

import jax
import jax.numpy as jnp
from jax import lax
from jax.experimental import pallas as pl
from jax.experimental.pallas import tpu as pltpu
from jax.experimental.pallas import tpu_sc as plsc

N_NODES = 10000
D = 128
NC = 2
NS = 16
NW = NC * NS
CHUNK = 125
IDXG = 16
ACC_ROWS = N_NODES


def _make_seg_kernel(nch: int):
    mesh = plsc.VectorSubcoreMesh(core_axis_name="c", subcore_axis_name="s")

    @functools.partial(
        pl.kernel,
        out_type=jax.ShapeDtypeStruct((NC, N_NODES, D), jnp.float32),
        mesh=mesh,
        scratch_types=[
            pltpu.VMEM((IDXG, CHUNK), jnp.int32),
            pltpu.VMEM((IDXG, CHUNK), jnp.int32),
            pltpu.VMEM((2, CHUNK, D), jnp.float32),
            pltpu.VMEM_SHARED((ACC_ROWS, D), jnp.float32),
            pltpu.SemaphoreType.DMA,
            pltpu.SemaphoreType.DMA,
        ],
    )
    def seg(x_hbm, edges_hbm, out_hbm, src_v, dst_v, rows_v, acc, sem0,
            sem1):
        cid = lax.axis_index("c")
        sid = lax.axis_index("s")
        wid = cid * NS + sid

        def zrow(r, carry):
            for c8 in range(D // 16):
                rows_v[0, r, pl.ds(c8 * 16, 16)] = jnp.zeros((16,), jnp.float32)
            return carry

        lax.fori_loop(0, CHUNK, zrow, 0)
        zpt = ACC_ROWS // NS
        for k in range(zpt // CHUNK):
            pltpu.sync_copy(rows_v.at[0],
                            acc.at[pl.ds(sid * zpt + k * CHUNK, CHUNK)])
        plsc.subcore_barrier()

        def start_gather(i, b, sem):
            pltpu.async_copy(x_hbm.at[src_v.at[i]], rows_v.at[b], sem)

        def wait_gather(i, b, sem):
            pltpu.make_async_copy(x_hbm.at[src_v.at[i]], rows_v.at[b], sem).wait()

        def group(g, carry):
            base = wid * nch + g * IDXG
            pltpu.sync_copy(edges_hbm.at[0, pl.ds(base, IDXG)], src_v)
            pltpu.sync_copy(edges_hbm.at[1, pl.ds(base, IDXG)], dst_v)
            start_gather(0, 0, sem0)

            def pair(k, c2):
                i0 = 2 * k
                start_gather(i0 + 1, 1, sem1)
                wait_gather(i0, 0, sem0)
                pltpu.sync_copy(rows_v.at[0], acc.at[dst_v.at[i0]], add=True)

                @pl.when(k < IDXG // 2 - 1)
                def _():
                    start_gather(i0 + 2, 0, sem0)

                wait_gather(i0 + 1, 1, sem1)
                pltpu.sync_copy(rows_v.at[1], acc.at[dst_v.at[i0 + 1]], add=True)
                return c2

            lax.fori_loop(0, IDXG // 2, pair, 0)
            return carry

        lax.fori_loop(0, nch // IDXG, group, 0)
        plsc.subcore_barrier()

        rpt = 624
        tail = N_NODES - (NS - 1) * rpt

        @pl.when(sid != NS - 1)
        def _():
            pltpu.sync_copy(acc.at[pl.ds(sid * rpt, rpt)],
                            out_hbm.at[cid, pl.ds(sid * rpt, rpt)])

        @pl.when(sid == NS - 1)
        def _():
            pltpu.sync_copy(acc.at[pl.ds((NS - 1) * rpt, tail)],
                            out_hbm.at[cid, pl.ds((NS - 1) * rpt, tail)])

    return seg


def _dense_mlp(partial, x, W_conv, W_self, bc2, W1, b12, W2, b22, W3t, b3s):
    RB = 2000
    grid = (N_NODES // RB,)

    def body(p_ref, x_ref, wc, ws, bc_r, w1, b1_r, w2, b2_r, w3t, b3_r, o_ref):
        agg = p_ref[0] + p_ref[1]
        h = agg @ wc[...] + x_ref[...] @ ws[...] + bc_r[...]
        h = jnp.maximum(h, 0.0)
        h = jnp.maximum(h @ w1[...] + b1_r[...], 0.0)
        h = jnp.maximum(h @ w2[...] + b2_r[...], 0.0)
        z = lax.dot_general(w3t[...], h, (((1,), (1,)), ((), ())),
                            preferred_element_type=jnp.float32)
        z = jnp.maximum(z + b3_r[...], 0.0)
        o_ref[...] = jax.nn.sigmoid(z).reshape(1, 1, RB)

    full = lambda i: (0, 0)
    return pl.pallas_call(
        body,
        grid=grid,
        in_specs=[
            pl.BlockSpec((NC, RB, D), lambda i: (0, i, 0)),
            pl.BlockSpec((RB, D), lambda i: (i, 0)),
            pl.BlockSpec((D, D), full),
            pl.BlockSpec((D, D), full),
            pl.BlockSpec((1, D), full),
            pl.BlockSpec((D, D), full),
            pl.BlockSpec((1, D), full),
            pl.BlockSpec((D, D), full),
            pl.BlockSpec((1, D), full),
            pl.BlockSpec((1, D), full),
            pl.BlockSpec((1, 1), full),
        ],
        out_specs=pl.BlockSpec((1, 1, RB), lambda i: (i, 0, 0)),
        out_shape=jax.ShapeDtypeStruct((N_NODES // RB, 1, RB), jnp.float32),
    )(partial, x, W_conv, W_self, bc2, W1, b12, W2, b22, W3t, b3s)


def kernel(x, edge_index, W_conv, W_self, b_conv, W1, b1, W2, b2, W3, b3):
    E = edge_index.shape[1]

    assert E % (NW * CHUNK) == 0, "edge count must divide into 125-edge chunks"
    nch = E // (NW * CHUNK)
    assert nch % IDXG == 0
    edges = edge_index.astype(jnp.int32).reshape(2, NW * nch, CHUNK)

    partial = _make_seg_kernel(nch)(x, edges)

    bc2 = b_conv.reshape(1, D)
    b12 = b1.reshape(1, D)
    b22 = b2.reshape(1, D)
    W3t = W3.reshape(1, D)
    b3s = b3.reshape(1, 1)

    out = _dense_mlp(partial, x, W_conv, W_self, bc2, W1, b12, W2, b22, W3t, b3s)
    return out.reshape(N_NODES)


# speedup vs baseline: 1.3721x; 1.0772x over previous
import functools

import jax
import jax.numpy as jnp
from jax import lax
from jax.experimental import pallas as pl
from jax.experimental.pallas import tpu as pltpu
from jax.experimental.pallas import tpu_sc as plsc

N_NODES = 10000
D = 128
NC = 2
NS = 16
NW = NC * NS
CHUNK = 125
IDXG = 8
ACC_ROWS = N_NODES


def _make_seg_kernel(nch: int):
    mesh = plsc.VectorSubcoreMesh(core_axis_name="c", subcore_axis_name="s")

    @functools.partial(
        pl.kernel,
        out_type=jax.ShapeDtypeStruct((NC, N_NODES, D), jnp.float32),
        mesh=mesh,
        scratch_types=[
            pltpu.VMEM((2 * IDXG, CHUNK), jnp.int32),
            pltpu.VMEM((2 * IDXG, CHUNK), jnp.int32),
            pltpu.VMEM((2, CHUNK, D), jnp.float32),
            pltpu.VMEM_SHARED((ACC_ROWS, D), jnp.float32),
            pltpu.SemaphoreType.DMA,
            pltpu.SemaphoreType.DMA,
            pltpu.SemaphoreType.DMA,
            pltpu.SemaphoreType.DMA,
        ],
    )
    def seg(x_hbm, edges_hbm, out_hbm, src_v, dst_v, rows_v, acc, sem0,
            sem1, isem0, isem1):
        cid = lax.axis_index("c")
        sid = lax.axis_index("s")
        wid = cid * NS + sid

        def zrow(r, carry):
            for c8 in range(D // 16):
                rows_v[0, r, pl.ds(c8 * 16, 16)] = jnp.zeros((16,), jnp.float32)
            return carry

        lax.fori_loop(0, CHUNK, zrow, 0)
        zpt = ACC_ROWS // NS
        for k in range(zpt // CHUNK):
            pltpu.sync_copy(rows_v.at[0],
                            acc.at[pl.ds(sid * zpt + k * CHUNK, CHUNK)])
        plsc.subcore_barrier()

        NH = nch // IDXG

        def start_gather(r, b, sem):
            pltpu.async_copy(x_hbm.at[src_v.at[r]], rows_v.at[b], sem)

        def wait_gather(r, b, sem):
            pltpu.make_async_copy(x_hbm.at[src_v.at[r]], rows_v.at[b], sem).wait()

        def scatter(r, b):
            pltpu.sync_copy(rows_v.at[b], acc.at[dst_v.at[r]], add=True)

        def load_idx_async(h, p):
            base = wid * nch + h * IDXG
            pltpu.async_copy(edges_hbm.at[0, pl.ds(base, IDXG)],
                             src_v.at[pl.ds(p * IDXG, IDXG)], isem0)
            pltpu.async_copy(edges_hbm.at[1, pl.ds(base, IDXG)],
                             dst_v.at[pl.ds(p * IDXG, IDXG)], isem1)

        def wait_idx(h, p):
            base = wid * nch + h * IDXG
            pltpu.make_async_copy(edges_hbm.at[0, pl.ds(base, IDXG)],
                                  src_v.at[pl.ds(p * IDXG, IDXG)], isem0).wait()
            pltpu.make_async_copy(edges_hbm.at[1, pl.ds(base, IDXG)],
                                  dst_v.at[pl.ds(p * IDXG, IDXG)], isem1).wait()

        pltpu.sync_copy(edges_hbm.at[0, pl.ds(wid * nch, IDXG)],
                        src_v.at[pl.ds(0, IDXG)])
        pltpu.sync_copy(edges_hbm.at[1, pl.ds(wid * nch, IDXG)],
                        dst_v.at[pl.ds(0, IDXG)])
        start_gather(0, 0, sem0)
        start_gather(1, 1, sem1)

        def super_(sg, carry):
            for p in (0, 1):
                h = 2 * sg + p

                @pl.when(h + 1 < NH)
                def _():
                    load_idx_async(h + 1, 1 - p)

                def pair(k, c2):
                    r0 = p * IDXG + 2 * k
                    wait_gather(r0, 0, sem0)
                    scatter(r0, 0)
                    start_gather(r0 + 2, 0, sem0)
                    wait_gather(r0 + 1, 1, sem1)
                    scatter(r0 + 1, 1)
                    start_gather(r0 + 3, 1, sem1)
                    return c2

                lax.fori_loop(0, IDXG // 2 - 1, pair, 0)

                rl = p * IDXG + IDXG - 2

                @pl.when(h + 1 < NH)
                def _():
                    wait_idx(h + 1, 1 - p)

                wait_gather(rl, 0, sem0)
                scatter(rl, 0)

                @pl.when(h + 1 < NH)
                def _():
                    start_gather((1 - p) * IDXG, 0, sem0)

                wait_gather(rl + 1, 1, sem1)
                scatter(rl + 1, 1)

                @pl.when(h + 1 < NH)
                def _():
                    start_gather((1 - p) * IDXG + 1, 1, sem1)

            return carry

        lax.fori_loop(0, NH // 2, super_, 0)
        plsc.subcore_barrier()

        rpt = 624
        tail = N_NODES - (NS - 1) * rpt

        @pl.when(sid != NS - 1)
        def _():
            pltpu.sync_copy(acc.at[pl.ds(sid * rpt, rpt)],
                            out_hbm.at[cid, pl.ds(sid * rpt, rpt)])

        @pl.when(sid == NS - 1)
        def _():
            pltpu.sync_copy(acc.at[pl.ds((NS - 1) * rpt, tail)],
                            out_hbm.at[cid, pl.ds((NS - 1) * rpt, tail)])

    return seg


def _dense_mlp(partial, x, W_conv, W_self, bc2, W1, b12, W2, b22, W3t, b3s):
    RB = 2000
    grid = (N_NODES // RB,)

    def body(p_ref, x_ref, wc, ws, bc_r, w1, b1_r, w2, b2_r, w3t, b3_r, o_ref):
        agg = p_ref[0] + p_ref[1]
        h = agg @ wc[...] + x_ref[...] @ ws[...] + bc_r[...]
        h = jnp.maximum(h, 0.0)
        h = jnp.maximum(h @ w1[...] + b1_r[...], 0.0)
        h = jnp.maximum(h @ w2[...] + b2_r[...], 0.0)
        z = lax.dot_general(w3t[...], h, (((1,), (1,)), ((), ())),
                            preferred_element_type=jnp.float32)
        z = jnp.maximum(z + b3_r[...], 0.0)
        o_ref[...] = jax.nn.sigmoid(z).reshape(1, 1, RB)

    full = lambda i: (0, 0)
    return pl.pallas_call(
        body,
        grid=grid,
        in_specs=[
            pl.BlockSpec((NC, RB, D), lambda i: (0, i, 0)),
            pl.BlockSpec((RB, D), lambda i: (i, 0)),
            pl.BlockSpec((D, D), full),
            pl.BlockSpec((D, D), full),
            pl.BlockSpec((1, D), full),
            pl.BlockSpec((D, D), full),
            pl.BlockSpec((1, D), full),
            pl.BlockSpec((D, D), full),
            pl.BlockSpec((1, D), full),
            pl.BlockSpec((1, D), full),
            pl.BlockSpec((1, 1), full),
        ],
        out_specs=pl.BlockSpec((1, 1, RB), lambda i: (i, 0, 0)),
        out_shape=jax.ShapeDtypeStruct((N_NODES // RB, 1, RB), jnp.float32),
    )(partial, x, W_conv, W_self, bc2, W1, b12, W2, b22, W3t, b3s)


def kernel(x, edge_index, W_conv, W_self, b_conv, W1, b1, W2, b2, W3, b3):
    E = edge_index.shape[1]

    assert E % (NW * CHUNK) == 0, "edge count must divide into 125-edge chunks"
    nch = E // (NW * CHUNK)
    assert nch % (2 * IDXG) == 0
    edges = edge_index.astype(jnp.int32).reshape(2, NW * nch, CHUNK)

    partial = _make_seg_kernel(nch)(x, edges)

    bc2 = b_conv.reshape(1, D)
    b12 = b1.reshape(1, D)
    b22 = b2.reshape(1, D)
    W3t = W3.reshape(1, D)
    b3s = b3.reshape(1, 1)

    out = _dense_mlp(partial, x, W_conv, W_self, bc2, W1, b12, W2, b22, W3t, b3s)
    return out.reshape(N_NODES)
